# baseline (speedup 1.0000x reference)
import functools

import jax
import jax.numpy as jnp
from jax import lax
from jax.experimental import pallas as pl
from jax.experimental.pallas import tpu as pltpu
from jax.experimental.pallas import tpu_sc as plsc

NUM_HEADS = 16
T = 1569
K = 10938
KPAD = 10944
FLAT = NUM_HEADS * KPAD

NC, NS, L = 2, 16, 16

HG = 4
NG = NUM_HEADS // HG
NSLICE = (NC * NS) // NG

NSLAB = 8
SLABR = KPAD // NSLAB
SLABE = SLABR * NUM_HEADS
SLABV = SLABR // L
SLABTAIL = SLABR - L

R = 8
NBLK = (T - 1) // R
NCHUNK = 25
NVEC = (T - 1) // L
TAIL = T - L


UNROLL = 7


def _gather_row(tbl_v, idx_v, val_v, r, base_k):
  def inner(c, c2):
    for u in range(UNROLL):
      off = (c * UNROLL + u) * L
      iv = idx_v[r, pl.ds(off, L)] + base_k
      val_v[r, pl.ds(off, L)] = plsc.load_gather(tbl_v, [iv])
    return c2

  lax.fori_loop(0, NVEC // UNROLL, inner, 0)
  iv = idx_v[r, pl.ds(TAIL, L)] + base_k
  val_v[r, pl.ds(TAIL, L)] = plsc.load_gather(tbl_v, [iv])


def _tec_body(tbl_hbm, idx_hbm, out_hbm, tbl_v, slab_v, idx0, idx1,
              val0, val1, si0, si1, sv0, sv1):
  wid = lax.axis_index("s") * NC + lax.axis_index("c")
  g = wid % NG
  sl = wid // NG
  h0 = HG * g

  lane16 = lax.iota(jnp.int32, L) * NUM_HEADS
  for si in range(NSLAB):
    pltpu.sync_copy(tbl_hbm.at[pl.ds(si * SLABE, SLABE)], slab_v)
    for hl in range(HG):

      def extract(c, c2, hl=hl, si=si):
        j0 = jnp.minimum(c * L, SLABTAIL)
        iv = lane16 + (j0 * NUM_HEADS + h0 + hl)
        tbl_v[pl.ds(hl * KPAD + si * SLABR + j0, L)] = plsc.load_gather(
            slab_v, [iv])
        return c2

      lax.fori_loop(0, SLABV + 1, extract, 0)

  blk0 = (49 * sl) // 2

  def row_base(k):
    return (blk0 + k) * R

  def process(k, idx_v, guard_first):
    base = row_base(k)
    for hl in range(HG):
      val_v = (val0, val1)[hl % 2]
      sem = (sv0, sv1)[hl % 2]
      dst = out_hbm.at[h0 + hl, pl.ds(base, R), :]
      if hl >= 2 or not guard_first:
        pltpu.make_async_copy(val_v, dst, sem).wait()
      else:

        @pl.when(k >= 1)
        def _():
          pltpu.make_async_copy(val_v, dst, sem).wait()

      for r in range(R):
        _gather_row(tbl_v, idx_v, val_v, r, hl * KPAD)
      pltpu.async_copy(val_v, dst, sem)

  pltpu.async_copy(idx_hbm.at[pl.ds(row_base(0), R), :], idx0, si0)

  def pair(j, carry):
    k = 2 * j
    pltpu.async_copy(idx_hbm.at[pl.ds(row_base(k + 1), R), :], idx1, si1)
    pltpu.make_async_copy(
        idx_hbm.at[pl.ds(row_base(k), R), :], idx0, si0).wait()
    process(k, idx0, True)
    pltpu.async_copy(idx_hbm.at[pl.ds(row_base(k + 2), R), :], idx0, si0)
    pltpu.make_async_copy(
        idx_hbm.at[pl.ds(row_base(k + 1), R), :], idx1, si1).wait()
    process(k + 1, idx1, True)
    return carry

  lax.fori_loop(0, (NCHUNK - 1) // 2, pair, 0)
  pltpu.make_async_copy(
      idx_hbm.at[pl.ds(row_base(NCHUNK - 1), R), :], idx0, si0).wait()
  process(NCHUNK - 1, idx0, False)

  base = row_base(NCHUNK - 1)
  for hl in (2, 3):
    pltpu.make_async_copy(
        (val0, val1)[hl % 2],
        out_hbm.at[h0 + hl, pl.ds(base, R), :], (sv0, sv1)[hl % 2]).wait()

  @pl.when(sl == NSLICE - 1)
  def _():
    pltpu.sync_copy(idx_hbm.at[pl.ds(NBLK * R, 1), :], idx0.at[pl.ds(0, 1), :])
    for hl in range(HG):
      _gather_row(tbl_v, idx0, val0, 0, hl * KPAD)
      pltpu.sync_copy(val0.at[pl.ds(0, 1), :],
                      out_hbm.at[h0 + hl, pl.ds(NBLK * R, 1), :])


_rpb_call = functools.partial(
    pl.kernel,
    out_type=jax.ShapeDtypeStruct((NUM_HEADS, T, T), jnp.float32),
    mesh=plsc.VectorSubcoreMesh(core_axis_name="c", subcore_axis_name="s"),
    scratch_types=[
        pltpu.VMEM((HG * KPAD,), jnp.float32),
        pltpu.VMEM((SLABE,), jnp.float32),
        pltpu.VMEM((R, T), jnp.int32),
        pltpu.VMEM((R, T), jnp.int32),
        pltpu.VMEM((R, T), jnp.float32),
        pltpu.VMEM((R, T), jnp.float32),
        pltpu.SemaphoreType.DMA,
        pltpu.SemaphoreType.DMA,
        pltpu.SemaphoreType.DMA,
        pltpu.SemaphoreType.DMA,
    ],
    compiler_params=pltpu.CompilerParams(needs_layout_passes=False),
)(_tec_body)


@jax.jit
def kernel(relative_position_bias_table, relative_position_index):
  tbl = relative_position_bias_table.astype(jnp.float32)
  tbl_flat = jnp.pad(tbl.reshape(-1), (0, FLAT - NUM_HEADS * K))
  idx = relative_position_index.astype(jnp.int32)
  return _rpb_call(tbl_flat, idx)


# speedup vs baseline: 2.4443x; 2.4443x over previous
import functools

import jax
import jax.numpy as jnp
from jax import lax
from jax.experimental import pallas as pl
from jax.experimental.pallas import tpu as pltpu
from jax.experimental.pallas import tpu_sc as plsc

NUM_HEADS = 16
T = 1569
K = 10938
KPAD = 10944
FLAT = NUM_HEADS * KPAD

NC, NS, L = 2, 16, 16

HG = 4
NG = NUM_HEADS // HG
NSLICE = (NC * NS) // NG

NSLAB = 8
SLABR = KPAD // NSLAB
SLABE = SLABR * NUM_HEADS
SLABV = SLABR // L
SLABTAIL = SLABR - L

R = 8
NBLK = (T - 1) // R
NCHUNK = 25
NVEC = (T - 1) // L
TAIL = T - L


def _gather_row(tbl_v, idx_v, val_v, r, base_k):
  @plsc.parallel_loop(0, NVEC * L, step=L, unroll=2)
  def inner(off):
    iv = idx_v[r, pl.ds(off, L)] + base_k
    val_v[r, pl.ds(off, L)] = plsc.load_gather(tbl_v, [iv])

  iv = idx_v[r, pl.ds(TAIL, L)] + base_k
  val_v[r, pl.ds(TAIL, L)] = plsc.load_gather(tbl_v, [iv])


def _tec_body(tbl_hbm, idx_hbm, out_hbm, tbl_v, slab_v, idx0, idx1,
              val0, val1, si0, si1, sv0, sv1):
  wid = lax.axis_index("s") * NC + lax.axis_index("c")
  g = wid % NG
  sl = wid // NG
  h0 = HG * g

  lane16 = lax.iota(jnp.int32, L) * NUM_HEADS
  for si in range(NSLAB):
    pltpu.sync_copy(tbl_hbm.at[pl.ds(si * SLABE, SLABE)], slab_v)
    for hl in range(HG):

      def extract(c, c2, hl=hl, si=si):
        j0 = jnp.minimum(c * L, SLABTAIL)
        iv = lane16 + (j0 * NUM_HEADS + h0 + hl)
        tbl_v[pl.ds(hl * KPAD + si * SLABR + j0, L)] = plsc.load_gather(
            slab_v, [iv])
        return c2

      lax.fori_loop(0, SLABV + 1, extract, 0)

  blk0 = (49 * sl) // 2

  def row_base(k):
    return (blk0 + k) * R

  def process(k, idx_v, guard_first):
    base = row_base(k)
    for hl in range(HG):
      val_v = (val0, val1)[hl % 2]
      sem = (sv0, sv1)[hl % 2]
      dst = out_hbm.at[h0 + hl, pl.ds(base, R), :]
      if hl >= 2 or not guard_first:
        pltpu.make_async_copy(val_v, dst, sem).wait()
      else:

        @pl.when(k >= 1)
        def _():
          pltpu.make_async_copy(val_v, dst, sem).wait()

      for r in range(R):
        _gather_row(tbl_v, idx_v, val_v, r, hl * KPAD)
      pltpu.async_copy(val_v, dst, sem)

  pltpu.async_copy(idx_hbm.at[pl.ds(row_base(0), R), :], idx0, si0)

  def pair(j, carry):
    k = 2 * j
    pltpu.async_copy(idx_hbm.at[pl.ds(row_base(k + 1), R), :], idx1, si1)
    pltpu.make_async_copy(
        idx_hbm.at[pl.ds(row_base(k), R), :], idx0, si0).wait()
    process(k, idx0, True)
    pltpu.async_copy(idx_hbm.at[pl.ds(row_base(k + 2), R), :], idx0, si0)
    pltpu.make_async_copy(
        idx_hbm.at[pl.ds(row_base(k + 1), R), :], idx1, si1).wait()
    process(k + 1, idx1, True)
    return carry

  lax.fori_loop(0, (NCHUNK - 1) // 2, pair, 0)
  pltpu.make_async_copy(
      idx_hbm.at[pl.ds(row_base(NCHUNK - 1), R), :], idx0, si0).wait()
  process(NCHUNK - 1, idx0, False)

  base = row_base(NCHUNK - 1)
  for hl in (2, 3):
    pltpu.make_async_copy(
        (val0, val1)[hl % 2],
        out_hbm.at[h0 + hl, pl.ds(base, R), :], (sv0, sv1)[hl % 2]).wait()

  @pl.when(sl == NSLICE - 1)
  def _():
    pltpu.sync_copy(idx_hbm.at[pl.ds(NBLK * R, 1), :], idx0.at[pl.ds(0, 1), :])
    for hl in range(HG):
      _gather_row(tbl_v, idx0, val0, 0, hl * KPAD)
      pltpu.sync_copy(val0.at[pl.ds(0, 1), :],
                      out_hbm.at[h0 + hl, pl.ds(NBLK * R, 1), :])


_rpb_call = functools.partial(
    pl.kernel,
    out_type=jax.ShapeDtypeStruct((NUM_HEADS, T, T), jnp.float32),
    mesh=plsc.VectorSubcoreMesh(core_axis_name="c", subcore_axis_name="s"),
    scratch_types=[
        pltpu.VMEM((HG * KPAD,), jnp.float32),
        pltpu.VMEM((SLABE,), jnp.float32),
        pltpu.VMEM((R, T), jnp.int32),
        pltpu.VMEM((R, T), jnp.int32),
        pltpu.VMEM((R, T), jnp.float32),
        pltpu.VMEM((R, T), jnp.float32),
        pltpu.SemaphoreType.DMA,
        pltpu.SemaphoreType.DMA,
        pltpu.SemaphoreType.DMA,
        pltpu.SemaphoreType.DMA,
    ],
    compiler_params=pltpu.CompilerParams(needs_layout_passes=False),
)(_tec_body)


@jax.jit
def kernel(relative_position_bias_table, relative_position_index):
  tbl = relative_position_bias_table.astype(jnp.float32)
  tbl_flat = jnp.pad(tbl.reshape(-1), (0, FLAT - NUM_HEADS * K))
  idx = relative_position_index.astype(jnp.int32)
  return _rpb_call(tbl_flat, idx)


# speedup vs baseline: 2.8607x; 1.1704x over previous
import functools

import jax
import jax.numpy as jnp
from jax import lax
from jax.experimental import pallas as pl
from jax.experimental.pallas import tpu as pltpu
from jax.experimental.pallas import tpu_sc as plsc

NUM_HEADS = 16
T = 1569
K = 10938
KPAD = 10944
FLAT = NUM_HEADS * KPAD

NC, NS, L = 2, 16, 16

HG = 4
NG = NUM_HEADS // HG
NSLICE = (NC * NS) // NG

NSLAB = 8
SLABR = KPAD // NSLAB
SLABE = SLABR * NUM_HEADS
SLABV = SLABR // L
SLABTAIL = SLABR - L

R = 8
NBLK = (T - 1) // R
NCHUNK = 25
NVEC = (T - 1) // L
TAIL = T - L


def _gather_row(tbl_v, idx_v, val_v, r, base_k):
  @plsc.parallel_loop(0, NVEC * L, step=L, unroll=7)
  def inner(off):
    iv = idx_v[r, pl.ds(off, L)] + base_k
    val_v[r, pl.ds(off, L)] = plsc.load_gather(tbl_v, [iv])

  iv = idx_v[r, pl.ds(TAIL, L)] + base_k
  val_v[r, pl.ds(TAIL, L)] = plsc.load_gather(tbl_v, [iv])


def _tec_body(tbl_hbm, idx_hbm, out_hbm, tbl_v, slab_v, idx0, idx1,
              val0, val1, si0, si1, sv0, sv1):
  wid = lax.axis_index("s") * NC + lax.axis_index("c")
  g = wid % NG
  sl = wid // NG
  h0 = HG * g

  lane16 = lax.iota(jnp.int32, L) * NUM_HEADS
  for si in range(NSLAB):
    pltpu.sync_copy(tbl_hbm.at[pl.ds(si * SLABE, SLABE)], slab_v)
    for hl in range(HG):

      def extract(c, c2, hl=hl, si=si):
        j0 = jnp.minimum(c * L, SLABTAIL)
        iv = lane16 + (j0 * NUM_HEADS + h0 + hl)
        tbl_v[pl.ds(hl * KPAD + si * SLABR + j0, L)] = plsc.load_gather(
            slab_v, [iv])
        return c2

      lax.fori_loop(0, SLABV + 1, extract, 0)

  blk0 = (49 * sl) // 2

  def row_base(k):
    return (blk0 + k) * R

  def process(k, idx_v, guard_first):
    base = row_base(k)
    for hl in range(HG):
      val_v = (val0, val1)[hl % 2]
      sem = (sv0, sv1)[hl % 2]
      dst = out_hbm.at[h0 + hl, pl.ds(base, R), :]
      if hl >= 2 or not guard_first:
        pltpu.make_async_copy(val_v, dst, sem).wait()
      else:

        @pl.when(k >= 1)
        def _():
          pltpu.make_async_copy(val_v, dst, sem).wait()

      for r in range(R):
        _gather_row(tbl_v, idx_v, val_v, r, hl * KPAD)
      pltpu.async_copy(val_v, dst, sem)

  pltpu.async_copy(idx_hbm.at[pl.ds(row_base(0), R), :], idx0, si0)

  def pair(j, carry):
    k = 2 * j
    pltpu.async_copy(idx_hbm.at[pl.ds(row_base(k + 1), R), :], idx1, si1)
    pltpu.make_async_copy(
        idx_hbm.at[pl.ds(row_base(k), R), :], idx0, si0).wait()
    process(k, idx0, True)
    pltpu.async_copy(idx_hbm.at[pl.ds(row_base(k + 2), R), :], idx0, si0)
    pltpu.make_async_copy(
        idx_hbm.at[pl.ds(row_base(k + 1), R), :], idx1, si1).wait()
    process(k + 1, idx1, True)
    return carry

  lax.fori_loop(0, (NCHUNK - 1) // 2, pair, 0)
  pltpu.make_async_copy(
      idx_hbm.at[pl.ds(row_base(NCHUNK - 1), R), :], idx0, si0).wait()
  process(NCHUNK - 1, idx0, False)

  base = row_base(NCHUNK - 1)
  for hl in (2, 3):
    pltpu.make_async_copy(
        (val0, val1)[hl % 2],
        out_hbm.at[h0 + hl, pl.ds(base, R), :], (sv0, sv1)[hl % 2]).wait()

  @pl.when(sl == NSLICE - 1)
  def _():
    pltpu.sync_copy(idx_hbm.at[pl.ds(NBLK * R, 1), :], idx0.at[pl.ds(0, 1), :])
    for hl in range(HG):
      _gather_row(tbl_v, idx0, val0, 0, hl * KPAD)
      pltpu.sync_copy(val0.at[pl.ds(0, 1), :],
                      out_hbm.at[h0 + hl, pl.ds(NBLK * R, 1), :])


_rpb_call = functools.partial(
    pl.kernel,
    out_type=jax.ShapeDtypeStruct((NUM_HEADS, T, T), jnp.float32),
    mesh=plsc.VectorSubcoreMesh(core_axis_name="c", subcore_axis_name="s"),
    scratch_types=[
        pltpu.VMEM((HG * KPAD,), jnp.float32),
        pltpu.VMEM((SLABE,), jnp.float32),
        pltpu.VMEM((R, T), jnp.int32),
        pltpu.VMEM((R, T), jnp.int32),
        pltpu.VMEM((R, T), jnp.float32),
        pltpu.VMEM((R, T), jnp.float32),
        pltpu.SemaphoreType.DMA,
        pltpu.SemaphoreType.DMA,
        pltpu.SemaphoreType.DMA,
        pltpu.SemaphoreType.DMA,
    ],
    compiler_params=pltpu.CompilerParams(needs_layout_passes=False),
)(_tec_body)


@jax.jit
def kernel(relative_position_bias_table, relative_position_index):
  tbl = relative_position_bias_table.astype(jnp.float32)
  tbl_flat = jnp.pad(tbl.reshape(-1), (0, FLAT - NUM_HEADS * K))
  idx = relative_position_index.astype(jnp.int32)
  return _rpb_call(tbl_flat, idx)


# speedup vs baseline: 7.2238x; 2.5252x over previous
import functools

import jax
import jax.numpy as jnp
from jax import lax
from jax.experimental import pallas as pl
from jax.experimental.pallas import tpu as pltpu
from jax.experimental.pallas import tpu_sc as plsc

NUM_HEADS = 16
T = 1569
K = 10938
KPAD = 10944
FLAT = NUM_HEADS * KPAD

NC, NS, L = 2, 16, 16

HG = 8
NG = NUM_HEADS // HG
NSLICE = (NC * NS) // NG

R = 8
NBLK = (T - 1) // R
NCHUNK = 13
NVEC = (T - 1) // L
TAIL = T - L


def _gather_row(tbl_v, idx_v, val_v, r):
  @plsc.parallel_loop(0, NVEC * L, step=L, unroll=7)
  def inner(off):
    iv = idx_v[r, pl.ds(off, L)]
    for hl in range(HG):
      val_v[0, hl, pl.ds(off, L)] = plsc.load_gather(tbl_v, [iv + hl * KPAD])

  iv = idx_v[r, pl.ds(TAIL, L)]
  for hl in range(HG):
    val_v[0, hl, pl.ds(TAIL, L)] = plsc.load_gather(tbl_v, [iv + hl * KPAD])


def _tec_body(tbl_hbm, idx_hbm, out_hbm, tbl_v, idx_v, val0, val1, sv0, sv1):
  wid = lax.axis_index("s") * NC + lax.axis_index("c")
  g = wid % NG
  sl = wid // NG
  h0 = HG * g

  for hl in range(HG):
    pltpu.sync_copy(tbl_hbm.at[pl.ds((h0 + hl) * KPAD, KPAD)],
                    tbl_v.at[pl.ds(hl * KPAD, KPAD)])

  blk0 = (49 * sl) // 4

  def block(blk, carry):
    base = (blk0 + blk) * R
    pltpu.sync_copy(idx_hbm.at[pl.ds(base, R), :], idx_v)
    for r in range(R):
      val_v = (val0, val1)[r % 2]
      sem = (sv0, sv1)[r % 2]
      dst = out_hbm.at[pl.ds(base + r, 1), pl.ds(h0, HG), :]
      if r < 2:

        @pl.when(blk >= 1)
        def _():
          pltpu.make_async_copy(val_v, dst, sem).wait()
      else:
        pltpu.make_async_copy(val_v, dst, sem).wait()

      _gather_row(tbl_v, idx_v, val_v, r)
      pltpu.async_copy(val_v, dst, sem)
    return carry

  lax.fori_loop(0, NCHUNK, block, 0)

  for r in (6, 7):
    row = (blk0 + NCHUNK - 1) * R + r
    pltpu.make_async_copy(
        (val0, val1)[r % 2],
        out_hbm.at[pl.ds(row, 1), pl.ds(h0, HG), :],
        (sv0, sv1)[r % 2]).wait()

  @pl.when(sl == NSLICE - 1)
  def _():
    pltpu.sync_copy(idx_hbm.at[pl.ds(NBLK * R, 1), :],
                    idx_v.at[pl.ds(0, 1), :])
    _gather_row(tbl_v, idx_v, val0, 0)
    pltpu.sync_copy(val0, out_hbm.at[pl.ds(NBLK * R, 1), pl.ds(h0, HG), :])


_rpb_call = functools.partial(
    pl.kernel,
    out_type=jax.ShapeDtypeStruct((T, NUM_HEADS, T), jnp.float32),
    mesh=plsc.VectorSubcoreMesh(core_axis_name="c", subcore_axis_name="s"),
    scratch_types=[
        pltpu.VMEM((HG * KPAD,), jnp.float32),
        pltpu.VMEM((R, T), jnp.int32),
        pltpu.VMEM((1, HG, T), jnp.float32),
        pltpu.VMEM((1, HG, T), jnp.float32),
        pltpu.SemaphoreType.DMA,
        pltpu.SemaphoreType.DMA,
    ],
    compiler_params=pltpu.CompilerParams(needs_layout_passes=False),
)(_tec_body)


@jax.jit
def kernel(relative_position_bias_table, relative_position_index):
  tbl = relative_position_bias_table.astype(jnp.float32)
  tbl_flat = jnp.pad(tbl, ((0, KPAD - K), (0, 0))).T.reshape(-1)
  idx = relative_position_index.astype(jnp.int32)
  out3 = _rpb_call(tbl_flat, idx)
  return jnp.transpose(out3, (1, 0, 2))


# speedup vs baseline: 7.3254x; 1.0141x over previous
import functools

import jax
import jax.numpy as jnp
from jax import lax
from jax.experimental import pallas as pl
from jax.experimental.pallas import tpu as pltpu
from jax.experimental.pallas import tpu_sc as plsc

NUM_HEADS = 16
T = 1569
K = 10938
KPAD = 10944
FLAT = NUM_HEADS * KPAD

NC, NS, L = 2, 16, 16

HG = 8
NG = NUM_HEADS // HG
NSLICE = (NC * NS) // NG

R = 8
NBLK = (T - 1) // R
NVEC = (T - 1) // L
TAIL = T - L


def _gather_row(tbl_v, idx_v, val_v, r):
  @plsc.parallel_loop(0, NVEC * L, step=L, unroll=7)
  def inner(off):
    iv = idx_v[r, pl.ds(off, L)]
    for hl in range(HG):
      val_v[0, hl, pl.ds(off, L)] = plsc.load_gather(tbl_v, [iv + hl * KPAD])

  iv = idx_v[r, pl.ds(TAIL, L)]
  for hl in range(HG):
    val_v[0, hl, pl.ds(TAIL, L)] = plsc.load_gather(tbl_v, [iv + hl * KPAD])


def _tec_body(tbl_hbm, idx_hbm, out_hbm, tbl_v, idx_v, val0, val1, sv0, sv1):
  wid = lax.axis_index("s") * NC + lax.axis_index("c")
  g = wid % NG
  sl = wid // NG
  h0 = HG * g

  for hl in range(HG):
    pltpu.sync_copy(tbl_hbm.at[pl.ds((h0 + hl) * KPAD, KPAD)],
                    tbl_v.at[pl.ds(hl * KPAD, KPAD)])

  blk0 = 12 * sl + jnp.minimum(sl, 4)
  nblocks = jnp.where(sl < 4, 13, 12)

  def block(blk, carry):
    base = (blk0 + blk) * R
    pltpu.sync_copy(idx_hbm.at[pl.ds(base, R), :], idx_v)
    for r in range(R):
      val_v = (val0, val1)[r % 2]
      sem = (sv0, sv1)[r % 2]
      dst = out_hbm.at[pl.ds(base + r, 1), pl.ds(h0, HG), :]
      if r < 2:

        @pl.when(blk >= 1)
        def _():
          pltpu.make_async_copy(val_v, dst, sem).wait()
      else:
        pltpu.make_async_copy(val_v, dst, sem).wait()

      _gather_row(tbl_v, idx_v, val_v, r)
      pltpu.async_copy(val_v, dst, sem)
    return carry

  lax.fori_loop(0, nblocks, block, 0)

  for r in (6, 7):
    row = (blk0 + nblocks - 1) * R + r
    pltpu.make_async_copy(
        (val0, val1)[r % 2],
        out_hbm.at[pl.ds(row, 1), pl.ds(h0, HG), :],
        (sv0, sv1)[r % 2]).wait()

  @pl.when(sl == NSLICE - 1)
  def _():
    pltpu.sync_copy(idx_hbm.at[pl.ds(NBLK * R, 1), :],
                    idx_v.at[pl.ds(0, 1), :])
    _gather_row(tbl_v, idx_v, val0, 0)
    pltpu.sync_copy(val0, out_hbm.at[pl.ds(NBLK * R, 1), pl.ds(h0, HG), :])


_rpb_call = functools.partial(
    pl.kernel,
    out_type=jax.ShapeDtypeStruct((T, NUM_HEADS, T), jnp.float32),
    mesh=plsc.VectorSubcoreMesh(core_axis_name="c", subcore_axis_name="s"),
    scratch_types=[
        pltpu.VMEM((HG * KPAD,), jnp.float32),
        pltpu.VMEM((R, T), jnp.int32),
        pltpu.VMEM((1, HG, T), jnp.float32),
        pltpu.VMEM((1, HG, T), jnp.float32),
        pltpu.SemaphoreType.DMA,
        pltpu.SemaphoreType.DMA,
    ],
    compiler_params=pltpu.CompilerParams(needs_layout_passes=False),
)(_tec_body)


@jax.jit
def kernel(relative_position_bias_table, relative_position_index):
  tbl = relative_position_bias_table.astype(jnp.float32)
  tbl_flat = jnp.pad(tbl, ((0, KPAD - K), (0, 0))).T.reshape(-1)
  idx = relative_position_index.astype(jnp.int32)
  out3 = _rpb_call(tbl_flat, idx)
  return jnp.transpose(out3, (1, 0, 2))
